# Initial kernel scaffold; baseline (speedup 1.0000x reference)
import jax
import jax.numpy as jnp
from jax.experimental import pallas as pl


def kernel(attn_bias, spatial_pos, spatial_pos_encoder, spatial_pos_encoder_rev):
    raise NotImplementedError("write your pallas kernel here")


# speedup vs baseline: 4.3064x; 4.3064x over previous
import functools

import jax
import jax.numpy as jnp
from jax import lax
from jax.experimental import pallas as pl
from jax.experimental.pallas import tpu as pltpu
from jax.experimental.pallas import tpu_sc as plsc

G = 4
N = 512
H = 32
S = 512

NC = 2
NS = 16
NW = NC * NS

ROWS = G * N
RPW = ROWS // NW
CHUNK = 4
LANES = 16


def _transpose_body(x_ref, o_ref):
    o_ref[0] = x_ref[0].T


def _transpose_gnn(x):
    B = 256
    return pl.pallas_call(
        _transpose_body,
        out_shape=jax.ShapeDtypeStruct((G, N, N), jnp.int32),
        grid=(G, N // B, N // B),
        in_specs=[pl.BlockSpec((1, B, B), lambda g, a, b: (g, b, a))],
        out_specs=pl.BlockSpec((1, B, B), lambda g, a, b: (g, a, b)),
    )(x)


def _sc_body(ab_hbm, sp_hbm, spt_hbm, enc_hbm, encr_hbm, out_hbm,
             enc_v, encr_v, sp_v, spt_v, ab_v, obuf_v):
    cid = lax.axis_index("c")
    sid = lax.axis_index("s")
    wid = sid * NC + cid

    pltpu.sync_copy(enc_hbm, enc_v)
    pltpu.sync_copy(encr_hbm, encr_v)

    row0 = wid * RPW
    g = row0 // N
    i_base = row0 % N

    @pl.loop(0, RPW // CHUNK)
    def _chunk(ck):
        i0 = i_base + ck * CHUNK
        pltpu.sync_copy(sp_hbm.at[g, pl.ds(i0, CHUNK)], sp_v)
        pltpu.sync_copy(spt_hbm.at[g, pl.ds(i0, CHUNK)], spt_v)
        pltpu.sync_copy(ab_hbm.at[g, pl.ds(i0, CHUNK)], ab_v)

        @pl.loop(0, CHUNK)
        def _row(c):
            @pl.loop(0, N // LANES)
            def _jvec(jv):
                sl = pl.ds(jv * LANES, LANES)
                spvec = sp_v[c, sl] * H
                sptvec = spt_v[c, sl] * H
                abvec = ab_v[c, sl]
                for h in range(H):
                    fwd = plsc.load_gather(enc_v, [spvec + h])
                    rev = plsc.load_gather(encr_v, [sptvec + h])
                    obuf_v[h, c, sl] = fwd + rev + abvec

        pltpu.sync_copy(obuf_v, out_hbm.at[g, :, pl.ds(i0, CHUNK), :])


@jax.jit
def kernel(attn_bias, spatial_pos, spatial_pos_encoder, spatial_pos_encoder_rev):
    spt = _transpose_gnn(spatial_pos)

    mesh = plsc.VectorSubcoreMesh(core_axis_name="c", subcore_axis_name="s")
    run = pl.kernel(
        _sc_body,
        out_type=jax.ShapeDtypeStruct((G, H, N, N), jnp.float32),
        mesh=mesh,
        compiler_params=pltpu.CompilerParams(needs_layout_passes=False),
        scratch_types=[
            pltpu.VMEM((S * H,), jnp.float32),
            pltpu.VMEM((S * H,), jnp.float32),
            pltpu.VMEM((CHUNK, N), jnp.int32),
            pltpu.VMEM((CHUNK, N), jnp.int32),
            pltpu.VMEM((CHUNK, N), jnp.float32),
            pltpu.VMEM((H, CHUNK, N), jnp.float32),
        ],
    )
    return run(attn_bias, spatial_pos, spt,
               spatial_pos_encoder.reshape(-1),
               spatial_pos_encoder_rev.reshape(-1))


# speedup vs baseline: 4.4885x; 1.0423x over previous
import functools

import jax
import jax.numpy as jnp
from jax import lax
from jax.experimental import pallas as pl
from jax.experimental.pallas import tpu as pltpu
from jax.experimental.pallas import tpu_sc as plsc

G = 4
N = 512
H = 32
S = 512

NC = 2
NS = 16
NW = NC * NS

ROWS = G * N
RPW = ROWS // NW
CHUNK = 2
LANES = 16


def _transpose_body(x_ref, o_ref):
    o_ref[0] = x_ref[0].T


def _transpose_gnn(x):
    B = 256
    return pl.pallas_call(
        _transpose_body,
        out_shape=jax.ShapeDtypeStruct((G, N, N), jnp.int32),
        grid=(G, N // B, N // B),
        in_specs=[pl.BlockSpec((1, B, B), lambda g, a, b: (g, b, a))],
        out_specs=pl.BlockSpec((1, B, B), lambda g, a, b: (g, a, b)),
    )(x)


NCHUNK = RPW // CHUNK


def _sc_body(ab_hbm, sp_hbm, spt_hbm, enc_hbm, encr_hbm, out_hbm,
             enc_v, encr_v, sp_v, spt_v, ab_v, obuf_v, sem_in, sem_out):
    cid = lax.axis_index("c")
    sid = lax.axis_index("s")
    wid = sid * NC + cid

    pltpu.sync_copy(enc_hbm, enc_v)
    pltpu.sync_copy(encr_hbm, encr_v)

    row0 = wid * RPW
    g = row0 // N
    i_base = row0 % N

    def istart(ck, par):
        i0 = i_base + ck * CHUNK
        pltpu.async_copy(sp_hbm.at[g, pl.ds(i0, CHUNK)], sp_v.at[par],
                         sem_in.at[par])
        pltpu.async_copy(spt_hbm.at[g, pl.ds(i0, CHUNK)], spt_v.at[par],
                         sem_in.at[par])
        pltpu.async_copy(ab_hbm.at[g, pl.ds(i0, CHUNK)], ab_v.at[par],
                         sem_in.at[par])

    def iwait(ck, par):
        i0 = i_base + ck * CHUNK
        pltpu.make_async_copy(sp_hbm.at[g, pl.ds(i0, CHUNK)], sp_v.at[par],
                              sem_in.at[par]).wait()
        pltpu.make_async_copy(spt_hbm.at[g, pl.ds(i0, CHUNK)], spt_v.at[par],
                              sem_in.at[par]).wait()
        pltpu.make_async_copy(ab_hbm.at[g, pl.ds(i0, CHUNK)], ab_v.at[par],
                              sem_in.at[par]).wait()

    def ostart(ck, par):
        i0 = i_base + ck * CHUNK
        pltpu.async_copy(obuf_v.at[par],
                         out_hbm.at[g, :, pl.ds(i0, CHUNK), :],
                         sem_out.at[par])

    def owait(ck, par):
        i0 = i_base + ck * CHUNK
        pltpu.make_async_copy(obuf_v.at[par],
                              out_hbm.at[g, :, pl.ds(i0, CHUNK), :],
                              sem_out.at[par]).wait()

    istart(0, 0)
    istart(1, 1)

    @pl.loop(0, NCHUNK, step=2)
    def _chunk(ck0):
        for par in range(2):
            ck = ck0 + par
            iwait(ck, par)

            @pl.when(ck >= 2)
            def _():
                owait(ck - 2, par)

            @pl.loop(0, CHUNK)
            def _row(c):
                @pl.loop(0, N // LANES)
                def _jvec(jv):
                    sl = pl.ds(jv * LANES, LANES)
                    spvec = sp_v[par, c, sl] * H
                    sptvec = spt_v[par, c, sl] * H
                    abvec = ab_v[par, c, sl]
                    for h in range(H):
                        fwd = plsc.load_gather(enc_v, [spvec + h])
                        rev = plsc.load_gather(encr_v, [sptvec + h])
                        obuf_v[par, h, c, sl] = fwd + rev + abvec

            ostart(ck, par)

            @pl.when(ck + 2 < NCHUNK)
            def _():
                istart(ck + 2, par)

    owait(NCHUNK - 2, 0)
    owait(NCHUNK - 1, 1)


@jax.jit
def kernel(attn_bias, spatial_pos, spatial_pos_encoder, spatial_pos_encoder_rev):
    spt = _transpose_gnn(spatial_pos)

    mesh = plsc.VectorSubcoreMesh(core_axis_name="c", subcore_axis_name="s")
    run = pl.kernel(
        _sc_body,
        out_type=jax.ShapeDtypeStruct((G, H, N, N), jnp.float32),
        mesh=mesh,
        compiler_params=pltpu.CompilerParams(needs_layout_passes=False),
        scratch_types=[
            pltpu.VMEM((S * H,), jnp.float32),
            pltpu.VMEM((S * H,), jnp.float32),
            pltpu.VMEM((2, CHUNK, N), jnp.int32),
            pltpu.VMEM((2, CHUNK, N), jnp.int32),
            pltpu.VMEM((2, CHUNK, N), jnp.float32),
            pltpu.VMEM((2, H, CHUNK, N), jnp.float32),
            pltpu.SemaphoreType.DMA((2,)),
            pltpu.SemaphoreType.DMA((2,)),
        ],
    )
    return run(attn_bias, spatial_pos, spt,
               spatial_pos_encoder.reshape(-1),
               spatial_pos_encoder_rev.reshape(-1))


# speedup vs baseline: 25.1524x; 5.6038x over previous
import jax
import jax.numpy as jnp
from jax import lax
from jax.experimental import pallas as pl
from jax.experimental.pallas import tpu as pltpu
from jax.experimental.pallas import tpu_sc as plsc

G = 4
N = 512
H = 32
S = 512

NC = 2
NS = 16
NW = NC * NS

ROWS = G * N
RPW = ROWS // NW
CHUNK = 2
LANES = 16
CN = CHUNK * N
NCHUNK = RPW // CHUNK
HP = H // 2


def _transpose_body(x_ref, o_ref):
    o_ref[0] = x_ref[0].T


def _transpose_gnn(x):
    B = 256
    return pl.pallas_call(
        _transpose_body,
        out_shape=jax.ShapeDtypeStruct((G, N, N), jnp.int32),
        grid=(G, N // B, N // B),
        in_specs=[pl.BlockSpec((1, B, B), lambda g, a, b: (g, b, a))],
        out_specs=pl.BlockSpec((1, B, B), lambda g, a, b: (g, a, b)),
    )(x)


def _pack_table(t):
    tb = t.astype(jnp.bfloat16).reshape(S, HP, 2)
    words = jax.lax.bitcast_convert_type(tb, jnp.int32)
    return words.T.reshape(HP * S)


def _sc_body(ab_hbm, sp_hbm, spt_hbm, encp_hbm, encrp_hbm, out_hbm,
             encp_v, encrp_v, sp_v, spt_v, ab_v, obuf_v, sem_in, sem_out):
    cid = lax.axis_index("c")
    sid = lax.axis_index("s")
    wid = sid * NC + cid

    pltpu.sync_copy(encp_hbm, encp_v)
    pltpu.sync_copy(encrp_hbm, encrp_v)

    row0 = wid * RPW
    g = row0 // N
    i_base = row0 % N

    def istart(ck, par):
        j0 = (i_base + ck * CHUNK) * N
        pltpu.async_copy(sp_hbm.at[g, pl.ds(j0, CN)], sp_v.at[par],
                         sem_in.at[par])
        pltpu.async_copy(spt_hbm.at[g, pl.ds(j0, CN)], spt_v.at[par],
                         sem_in.at[par])
        pltpu.async_copy(ab_hbm.at[g, pl.ds(j0, CN)], ab_v.at[par],
                         sem_in.at[par])

    def iwait(ck, par):
        j0 = (i_base + ck * CHUNK) * N
        pltpu.make_async_copy(sp_hbm.at[g, pl.ds(j0, CN)], sp_v.at[par],
                              sem_in.at[par]).wait()
        pltpu.make_async_copy(spt_hbm.at[g, pl.ds(j0, CN)], spt_v.at[par],
                              sem_in.at[par]).wait()
        pltpu.make_async_copy(ab_hbm.at[g, pl.ds(j0, CN)], ab_v.at[par],
                              sem_in.at[par]).wait()

    def ostart(ck, par):
        j0 = (i_base + ck * CHUNK) * N
        pltpu.async_copy(obuf_v.at[par], out_hbm.at[g, :, pl.ds(j0, CN)],
                         sem_out.at[par])

    def owait(ck, par):
        j0 = (i_base + ck * CHUNK) * N
        pltpu.make_async_copy(obuf_v.at[par],
                              out_hbm.at[g, :, pl.ds(j0, CN)],
                              sem_out.at[par]).wait()

    istart(0, 0)
    istart(1, 1)

    @pl.loop(0, NCHUNK, step=2)
    def _chunk(ck0):
        for par in range(2):
            ck = ck0 + par
            iwait(ck, par)

            @pl.when(ck >= 2)
            def _():
                owait(ck - 2, par)

            @pl.loop(0, CN // LANES)
            def _t(t):
                sl = pl.ds(t * LANES, LANES)
                spvec = sp_v[par, sl]
                sptvec = spt_v[par, sl]
                abvec = ab_v[par, sl]
                for k0 in range(0, HP, 4):
                    ks = range(k0, k0 + 4)
                    fwds = [plsc.load_gather(encp_v.at[pl.ds(k * S, S)],
                                             [spvec]) for k in ks]
                    revs = [plsc.load_gather(encrp_v.at[pl.ds(k * S, S)],
                                             [sptvec]) for k in ks]
                    for u, k in enumerate(ks):
                        pair = (plsc.bitcast(fwds[u], jnp.bfloat16)
                                + plsc.bitcast(revs[u], jnp.bfloat16))
                        lo, hi = plsc.unpack(
                            pair, format=plsc.PackFormat.INTERLEAVED)
                        obuf_v[par, 2 * k, sl] = lo + abvec
                        obuf_v[par, 2 * k + 1, sl] = hi + abvec

            ostart(ck, par)

            @pl.when(ck + 2 < NCHUNK)
            def _():
                istart(ck + 2, par)

    owait(NCHUNK - 2, 0)
    owait(NCHUNK - 1, 1)


@jax.jit
def kernel(attn_bias, spatial_pos, spatial_pos_encoder, spatial_pos_encoder_rev):
    spt = _transpose_gnn(spatial_pos)

    mesh = plsc.VectorSubcoreMesh(core_axis_name="c", subcore_axis_name="s")
    run = pl.kernel(
        _sc_body,
        out_type=jax.ShapeDtypeStruct((G, H, N * N), jnp.float32),
        mesh=mesh,
        compiler_params=pltpu.CompilerParams(needs_layout_passes=False),
        scratch_types=[
            pltpu.VMEM((HP * S,), jnp.int32),
            pltpu.VMEM((HP * S,), jnp.int32),
            pltpu.VMEM((2, CN), jnp.int32),
            pltpu.VMEM((2, CN), jnp.int32),
            pltpu.VMEM((2, CN), jnp.float32),
            pltpu.VMEM((2, H, CN), jnp.float32),
            pltpu.SemaphoreType.DMA((2,)),
            pltpu.SemaphoreType.DMA((2,)),
        ],
    )
    out = run(attn_bias.reshape(G, N * N), spatial_pos.reshape(G, N * N),
              spt.reshape(G, N * N),
              _pack_table(spatial_pos_encoder),
              _pack_table(spatial_pos_encoder_rev))
    return out.reshape(G, H, N, N)


# speedup vs baseline: 27.2941x; 1.0851x over previous
import jax
import jax.numpy as jnp
from jax import lax
from jax.experimental import pallas as pl
from jax.experimental.pallas import tpu as pltpu
from jax.experimental.pallas import tpu_sc as plsc

G = 4
N = 512
H = 32
S = 512

NC = 2
NS = 16
NW = NC * NS

ROWS = G * N
RPW = ROWS // NW
CHUNK = 2
LANES = 16
CN = CHUNK * N
NCHUNK = RPW // CHUNK
HP = H // 2


def _transpose_body(x_ref, o_ref):
    o_ref[0] = x_ref[0].T


def _transpose_gnn(x):
    B = 256
    return pl.pallas_call(
        _transpose_body,
        out_shape=jax.ShapeDtypeStruct((G, N, N), jnp.int32),
        grid=(G, N // B, N // B),
        in_specs=[pl.BlockSpec((1, B, B), lambda g, a, b: (g, b, a))],
        out_specs=pl.BlockSpec((1, B, B), lambda g, a, b: (g, a, b)),
    )(x)


def _pack_table(t):
    tb = t.astype(jnp.bfloat16).reshape(S, HP, 2)
    words = jax.lax.bitcast_convert_type(tb, jnp.int32)
    return words.T.reshape(HP * S)


def _sc_body(ab_hbm, sp_hbm, spt_hbm, encp_hbm, encrp_hbm, out_hbm,
             encp_v, encrp_v, sp_v, spt_v, ab_v, obuf_v, sem_in, sem_out):
    cid = lax.axis_index("c")
    sid = lax.axis_index("s")
    wid = sid * NC + cid

    pltpu.sync_copy(encp_hbm, encp_v)
    pltpu.sync_copy(encrp_hbm, encrp_v)

    row0 = wid * RPW
    g = row0 // N
    i_base = row0 % N

    def istart(ck, par):
        j0 = (i_base + ck * CHUNK) * N
        pltpu.async_copy(sp_hbm.at[g, pl.ds(j0, CN)], sp_v.at[par],
                         sem_in.at[par])
        pltpu.async_copy(spt_hbm.at[g, pl.ds(j0, CN)], spt_v.at[par],
                         sem_in.at[par])
        pltpu.async_copy(ab_hbm.at[g, pl.ds(j0, CN)], ab_v.at[par],
                         sem_in.at[par])

    def iwait(ck, par):
        j0 = (i_base + ck * CHUNK) * N
        pltpu.make_async_copy(sp_hbm.at[g, pl.ds(j0, CN)], sp_v.at[par],
                              sem_in.at[par]).wait()
        pltpu.make_async_copy(spt_hbm.at[g, pl.ds(j0, CN)], spt_v.at[par],
                              sem_in.at[par]).wait()
        pltpu.make_async_copy(ab_hbm.at[g, pl.ds(j0, CN)], ab_v.at[par],
                              sem_in.at[par]).wait()

    def ostart(ck, par):
        j0 = (i_base + ck * CHUNK) * N
        pltpu.async_copy(obuf_v.at[par], out_hbm.at[g, :, pl.ds(j0, CN)],
                         sem_out.at[par])

    def owait(ck, par):
        j0 = (i_base + ck * CHUNK) * N
        pltpu.make_async_copy(obuf_v.at[par],
                              out_hbm.at[g, :, pl.ds(j0, CN)],
                              sem_out.at[par]).wait()

    istart(0, 0)
    istart(1, 1)

    @pl.loop(0, NCHUNK, step=2)
    def _chunk(ck0):
        for par in range(2):
            ck = ck0 + par
            iwait(ck, par)

            @pl.when(ck >= 2)
            def _():
                owait(ck - 2, par)

            GRP = 4

            @pl.loop(0, CN // LANES, unroll=2)
            def _t(t):
                sl = pl.ds(t * LANES, LANES)
                spvec = sp_v[par, sl]
                sptvec = spt_v[par, sl]
                abvec = ab_v[par, sl]

                def gathers(k0):
                    ks = range(k0, k0 + GRP)
                    fwds = [plsc.load_gather(encp_v.at[pl.ds(k * S, S)],
                                             [spvec]) for k in ks]
                    revs = [plsc.load_gather(encrp_v.at[pl.ds(k * S, S)],
                                             [sptvec]) for k in ks]
                    return fwds, revs

                def arith(k0, fwds, revs):
                    for u, k in enumerate(range(k0, k0 + GRP)):
                        pair = (plsc.bitcast(fwds[u], jnp.bfloat16)
                                + plsc.bitcast(revs[u], jnp.bfloat16))
                        lo, hi = plsc.unpack(
                            pair, format=plsc.PackFormat.INTERLEAVED)
                        obuf_v[par, 2 * k, sl] = lo + abvec
                        obuf_v[par, 2 * k + 1, sl] = hi + abvec

                pend = gathers(0)
                for k0 in range(GRP, HP, GRP):
                    cur = gathers(k0)
                    arith(k0 - GRP, *pend)
                    pend = cur
                arith(HP - GRP, *pend)

            ostart(ck, par)

            @pl.when(ck + 2 < NCHUNK)
            def _():
                istart(ck + 2, par)

    owait(NCHUNK - 2, 0)
    owait(NCHUNK - 1, 1)


@jax.jit
def kernel(attn_bias, spatial_pos, spatial_pos_encoder, spatial_pos_encoder_rev):
    spt = _transpose_gnn(spatial_pos)

    mesh = plsc.VectorSubcoreMesh(core_axis_name="c", subcore_axis_name="s")
    run = pl.kernel(
        _sc_body,
        out_type=jax.ShapeDtypeStruct((G, H, N * N), jnp.float32),
        mesh=mesh,
        compiler_params=pltpu.CompilerParams(needs_layout_passes=False),
        scratch_types=[
            pltpu.VMEM((HP * S,), jnp.int32),
            pltpu.VMEM((HP * S,), jnp.int32),
            pltpu.VMEM((2, CN), jnp.int32),
            pltpu.VMEM((2, CN), jnp.int32),
            pltpu.VMEM((2, CN), jnp.float32),
            pltpu.VMEM((2, H, CN), jnp.float32),
            pltpu.SemaphoreType.DMA((2,)),
            pltpu.SemaphoreType.DMA((2,)),
        ],
    )
    out = run(attn_bias.reshape(G, N * N), spatial_pos.reshape(G, N * N),
              spt.reshape(G, N * N),
              _pack_table(spatial_pos_encoder),
              _pack_table(spatial_pos_encoder_rev))
    return out.reshape(G, H, N, N)


# speedup vs baseline: 28.3010x; 1.0369x over previous
import jax
import jax.numpy as jnp
from jax import lax
from jax.experimental import pallas as pl
from jax.experimental.pallas import tpu as pltpu
from jax.experimental.pallas import tpu_sc as plsc

G = 4
N = 512
H = 32
S = 512

NC = 2
NS = 16
NW = NC * NS

ROWS = G * N
RPW = ROWS // NW
CHUNK = 2
LANES = 16
CN = CHUNK * N
NCHUNK = RPW // CHUNK
HP = H // 2


def _transpose_body(x_ref, o_ref):
    o_ref[0] = x_ref[0].T


def _transpose_gnn(x):
    B = 256
    return pl.pallas_call(
        _transpose_body,
        out_shape=jax.ShapeDtypeStruct((G, N, N), jnp.int32),
        grid=(G, N // B, N // B),
        in_specs=[pl.BlockSpec((1, B, B), lambda g, a, b: (g, b, a))],
        out_specs=pl.BlockSpec((1, B, B), lambda g, a, b: (g, a, b)),
    )(x)


def _pack_table(t):
    tb = t.astype(jnp.bfloat16).reshape(S, HP, 2)
    words = jax.lax.bitcast_convert_type(tb, jnp.int32)
    return words.T.reshape(HP * S)


def _sc_body(ab_hbm, sp_hbm, spt_hbm, encp_hbm, encrp_hbm, out_hbm,
             encp_v, encrp_v, sp_v, spt_v, ab_v, obuf_v, sem_in, sem_out):
    cid = lax.axis_index("c")
    sid = lax.axis_index("s")
    wid = sid * NC + cid

    pltpu.sync_copy(encp_hbm, encp_v)
    pltpu.sync_copy(encrp_hbm, encrp_v)

    row0 = wid * RPW
    g = row0 // N
    i_base = row0 % N

    def istart(ck, par):
        j0 = (i_base + ck * CHUNK) * N
        pltpu.async_copy(sp_hbm.at[g, pl.ds(j0, CN)], sp_v.at[par],
                         sem_in.at[par])
        pltpu.async_copy(spt_hbm.at[g, pl.ds(j0, CN)], spt_v.at[par],
                         sem_in.at[par])
        pltpu.async_copy(ab_hbm.at[g, pl.ds(j0, CN)], ab_v.at[par],
                         sem_in.at[par])

    def iwait(ck, par):
        j0 = (i_base + ck * CHUNK) * N
        pltpu.make_async_copy(sp_hbm.at[g, pl.ds(j0, CN)], sp_v.at[par],
                              sem_in.at[par]).wait()
        pltpu.make_async_copy(spt_hbm.at[g, pl.ds(j0, CN)], spt_v.at[par],
                              sem_in.at[par]).wait()
        pltpu.make_async_copy(ab_hbm.at[g, pl.ds(j0, CN)], ab_v.at[par],
                              sem_in.at[par]).wait()

    def ostart(ck, par):
        j0 = (i_base + ck * CHUNK) * N
        pltpu.async_copy(obuf_v.at[par], out_hbm.at[g, :, pl.ds(j0, CN)],
                         sem_out.at[par])

    def owait(ck, par):
        j0 = (i_base + ck * CHUNK) * N
        pltpu.make_async_copy(obuf_v.at[par],
                              out_hbm.at[g, :, pl.ds(j0, CN)],
                              sem_out.at[par]).wait()

    istart(0, 0)
    istart(1, 1)

    @pl.loop(0, NCHUNK, step=2)
    def _chunk(ck0):
        for par in range(2):
            ck = ck0 + par
            iwait(ck, par)

            @pl.when(ck >= 2)
            def _():
                owait(ck - 2, par)

            GRP = 4

            @plsc.parallel_loop(0, CN // LANES, unroll=2)
            def _t(t):
                sl = pl.ds(t * LANES, LANES)
                spvec = sp_v[par, sl]
                sptvec = spt_v[par, sl]
                abvec = ab_v[par, sl]

                def gathers(k0):
                    ks = range(k0, k0 + GRP)
                    fwds = [plsc.load_gather(encp_v.at[pl.ds(k * S, S)],
                                             [spvec]) for k in ks]
                    revs = [plsc.load_gather(encrp_v.at[pl.ds(k * S, S)],
                                             [sptvec]) for k in ks]
                    return fwds, revs

                def arith(k0, fwds, revs):
                    for u, k in enumerate(range(k0, k0 + GRP)):
                        pair = (plsc.bitcast(fwds[u], jnp.bfloat16)
                                + plsc.bitcast(revs[u], jnp.bfloat16))
                        lo, hi = plsc.unpack(
                            pair, format=plsc.PackFormat.INTERLEAVED)
                        obuf_v[par, 2 * k, sl] = lo + abvec
                        obuf_v[par, 2 * k + 1, sl] = hi + abvec

                pend = gathers(0)
                for k0 in range(GRP, HP, GRP):
                    cur = gathers(k0)
                    arith(k0 - GRP, *pend)
                    pend = cur
                arith(HP - GRP, *pend)

            ostart(ck, par)

            @pl.when(ck + 2 < NCHUNK)
            def _():
                istart(ck + 2, par)

    owait(NCHUNK - 2, 0)
    owait(NCHUNK - 1, 1)


@jax.jit
def kernel(attn_bias, spatial_pos, spatial_pos_encoder, spatial_pos_encoder_rev):
    spt = _transpose_gnn(spatial_pos)

    mesh = plsc.VectorSubcoreMesh(core_axis_name="c", subcore_axis_name="s")
    run = pl.kernel(
        _sc_body,
        out_type=jax.ShapeDtypeStruct((G, H, N * N), jnp.float32),
        mesh=mesh,
        compiler_params=pltpu.CompilerParams(needs_layout_passes=False),
        scratch_types=[
            pltpu.VMEM((HP * S,), jnp.int32),
            pltpu.VMEM((HP * S,), jnp.int32),
            pltpu.VMEM((2, CN), jnp.int32),
            pltpu.VMEM((2, CN), jnp.int32),
            pltpu.VMEM((2, CN), jnp.float32),
            pltpu.VMEM((2, H, CN), jnp.float32),
            pltpu.SemaphoreType.DMA((2,)),
            pltpu.SemaphoreType.DMA((2,)),
        ],
    )
    out = run(attn_bias.reshape(G, N * N), spatial_pos.reshape(G, N * N),
              spt.reshape(G, N * N),
              _pack_table(spatial_pos_encoder),
              _pack_table(spatial_pos_encoder_rev))
    return out.reshape(G, H, N, N)


# speedup vs baseline: 41.2353x; 1.4570x over previous
import jax
import jax.numpy as jnp
from jax import lax
from jax.experimental import pallas as pl
from jax.experimental.pallas import tpu as pltpu
from jax.experimental.pallas import tpu_sc as plsc

G = 4
N = 512
H = 32
S = 512

NC = 2
NS = 16
NW = NC * NS

ROWS = G * N
RPW = ROWS // NW
CHUNK = 2
LANES = 16
CN = CHUNK * N
NCHUNK = RPW // CHUNK
HP = H // 2


def _transpose_body(x_ref, o_ref):
    o_ref[0] = x_ref[0].T


def _transpose_gnn(x):
    B = 256
    return pl.pallas_call(
        _transpose_body,
        out_shape=jax.ShapeDtypeStruct((G, N, N), jnp.int32),
        grid=(G, N // B, N // B),
        in_specs=[pl.BlockSpec((1, B, B), lambda g, a, b: (g, b, a))],
        out_specs=pl.BlockSpec((1, B, B), lambda g, a, b: (g, a, b)),
    )(x)


def _pack_table(t):
    tb = t.astype(jnp.bfloat16).reshape(S, HP, 2)
    words = jax.lax.bitcast_convert_type(tb, jnp.int32)
    return words.T.reshape(HP * S)


def _sc_body(ab_hbm, sp_hbm, spt_hbm, encp_hbm, encrp_hbm, out_hbm,
             encp_v, encrp_v, sp_v, spt_v, ab_v, obuf_v, sem_in, sem_out):
    cid = lax.axis_index("c")
    sid = lax.axis_index("s")
    wid = sid * NC + cid

    pltpu.sync_copy(encp_hbm, encp_v)
    pltpu.sync_copy(encrp_hbm, encrp_v)

    row0 = wid * RPW
    g = row0 // N
    i_base = row0 % N

    def istart(ck, par):
        i0 = i_base + ck * CHUNK
        pltpu.async_copy(sp_hbm.at[g, pl.ds(i0, CHUNK), :], sp_v.at[par],
                         sem_in.at[par])
        pltpu.async_copy(spt_hbm.at[g, pl.ds(i0, CHUNK), :], spt_v.at[par],
                         sem_in.at[par])
        pltpu.async_copy(ab_hbm.at[g, pl.ds(i0, CHUNK), :], ab_v.at[par],
                         sem_in.at[par])

    def iwait(ck, par):
        i0 = i_base + ck * CHUNK
        pltpu.make_async_copy(sp_hbm.at[g, pl.ds(i0, CHUNK), :],
                              sp_v.at[par], sem_in.at[par]).wait()
        pltpu.make_async_copy(spt_hbm.at[g, pl.ds(i0, CHUNK), :],
                              spt_v.at[par], sem_in.at[par]).wait()
        pltpu.make_async_copy(ab_hbm.at[g, pl.ds(i0, CHUNK), :],
                              ab_v.at[par], sem_in.at[par]).wait()

    def ostart(ck, par):
        i0 = i_base + ck * CHUNK
        pltpu.async_copy(obuf_v.at[par],
                         out_hbm.at[g, :, pl.ds(i0, CHUNK), :],
                         sem_out.at[par])

    def owait(ck, par):
        i0 = i_base + ck * CHUNK
        pltpu.make_async_copy(obuf_v.at[par],
                              out_hbm.at[g, :, pl.ds(i0, CHUNK), :],
                              sem_out.at[par]).wait()

    istart(0, 0)
    istart(1, 1)

    @pl.loop(0, NCHUNK, step=2)
    def _chunk(ck0):
        for par in range(2):
            ck = ck0 + par
            iwait(ck, par)

            @pl.when(ck >= 2)
            def _():
                owait(ck - 2, par)

            GRP = 4

            for c in range(CHUNK):
                @plsc.parallel_loop(0, N // LANES, unroll=2)
                def _t(t):
                    sl = pl.ds(t * LANES, LANES)
                    spvec = sp_v[par, c, sl]
                    sptvec = spt_v[par, c, sl]
                    abvec = ab_v[par, c, sl]

                    def gathers(k0):
                        ks = range(k0, k0 + GRP)
                        fwds = [plsc.load_gather(encp_v.at[pl.ds(k * S, S)],
                                                 [spvec]) for k in ks]
                        revs = [plsc.load_gather(encrp_v.at[pl.ds(k * S, S)],
                                                 [sptvec]) for k in ks]
                        return fwds, revs

                    def arith(k0, fwds, revs):
                        for u, k in enumerate(range(k0, k0 + GRP)):
                            pair = (plsc.bitcast(fwds[u], jnp.bfloat16)
                                    + plsc.bitcast(revs[u], jnp.bfloat16))
                            lo, hi = plsc.unpack(
                                pair, format=plsc.PackFormat.INTERLEAVED)
                            obuf_v[par, 2 * k, c, sl] = lo + abvec
                            obuf_v[par, 2 * k + 1, c, sl] = hi + abvec

                    pend = gathers(0)
                    for k0 in range(GRP, HP, GRP):
                        cur = gathers(k0)
                        arith(k0 - GRP, *pend)
                        pend = cur
                    arith(HP - GRP, *pend)

            ostart(ck, par)

            @pl.when(ck + 2 < NCHUNK)
            def _():
                istart(ck + 2, par)

    owait(NCHUNK - 2, 0)
    owait(NCHUNK - 1, 1)


@jax.jit
def kernel(attn_bias, spatial_pos, spatial_pos_encoder, spatial_pos_encoder_rev):
    spt = _transpose_gnn(spatial_pos)

    mesh = plsc.VectorSubcoreMesh(core_axis_name="c", subcore_axis_name="s")
    run = pl.kernel(
        _sc_body,
        out_type=jax.ShapeDtypeStruct((G, H, N, N), jnp.float32),
        mesh=mesh,
        compiler_params=pltpu.CompilerParams(needs_layout_passes=False),
        scratch_types=[
            pltpu.VMEM((HP * S,), jnp.int32),
            pltpu.VMEM((HP * S,), jnp.int32),
            pltpu.VMEM((2, CHUNK, N), jnp.int32),
            pltpu.VMEM((2, CHUNK, N), jnp.int32),
            pltpu.VMEM((2, CHUNK, N), jnp.float32),
            pltpu.VMEM((2, H, CHUNK, N), jnp.float32),

            pltpu.SemaphoreType.DMA((2,)),
            pltpu.SemaphoreType.DMA((2,)),
        ],
    )
    return run(attn_bias, spatial_pos, spt,
               _pack_table(spatial_pos_encoder),
               _pack_table(spatial_pos_encoder_rev))


# speedup vs baseline: 66.9494x; 1.6236x over previous
import jax
import jax.numpy as jnp
from jax import lax
from jax.experimental import pallas as pl
from jax.experimental.pallas import tpu as pltpu
from jax.experimental.pallas import tpu_sc as plsc

G = 4
N = 512
H = 32
S = 512

NC = 2
NS = 16
NW = NC * NS

ROWS = G * N
RPW = ROWS // NW
CHUNK = 2
LANES = 16
CN = CHUNK * N
NCHUNK = RPW // CHUNK
HP = H // 2


def _transpose_body(x_ref, o_ref):
    o_ref[0] = x_ref[0].T


def _transpose_gnn(x):
    B = 256
    return pl.pallas_call(
        _transpose_body,
        out_shape=jax.ShapeDtypeStruct((G, N, N), jnp.int32),
        grid=(G, N // B, N // B),
        in_specs=[pl.BlockSpec((1, B, B), lambda g, a, b: (g, b, a))],
        out_specs=pl.BlockSpec((1, B, B), lambda g, a, b: (g, a, b)),
    )(x)


def _pack_table(t):
    tb = t.astype(jnp.bfloat16).reshape(S, HP, 2)
    words = jax.lax.bitcast_convert_type(tb, jnp.int32)
    return words.T.reshape(HP * S)


def _sc_body(ab_hbm, sp_hbm, spt_hbm, encp_hbm, encrp_hbm, out_hbm,
             encp_v, encrp_v, sp_v, spt_v, ab_v, obuf_v, sem_in, sem_out):
    cid = lax.axis_index("c")
    sid = lax.axis_index("s")
    wid = sid * NC + cid

    pltpu.sync_copy(encp_hbm, encp_v)
    pltpu.sync_copy(encrp_hbm, encrp_v)

    row0 = wid * RPW
    g = row0 // N
    i_base = row0 % N

    def istart(ck, par):
        i0 = i_base + ck * CHUNK
        pltpu.async_copy(sp_hbm.at[g, pl.ds(i0, CHUNK), :], sp_v.at[par],
                         sem_in.at[par])
        pltpu.async_copy(spt_hbm.at[g, pl.ds(i0, CHUNK), :], spt_v.at[par],
                         sem_in.at[par])
        pltpu.async_copy(ab_hbm.at[g, pl.ds(i0, CHUNK), :], ab_v.at[par],
                         sem_in.at[par])

    def iwait(ck, par):
        i0 = i_base + ck * CHUNK
        pltpu.make_async_copy(sp_hbm.at[g, pl.ds(i0, CHUNK), :],
                              sp_v.at[par], sem_in.at[par]).wait()
        pltpu.make_async_copy(spt_hbm.at[g, pl.ds(i0, CHUNK), :],
                              spt_v.at[par], sem_in.at[par]).wait()
        pltpu.make_async_copy(ab_hbm.at[g, pl.ds(i0, CHUNK), :],
                              ab_v.at[par], sem_in.at[par]).wait()

    def ostart(ck, par):
        i0 = i_base + ck * CHUNK
        pltpu.async_copy(obuf_v.at[par],
                         out_hbm.at[g, :, pl.ds(i0, CHUNK), :],
                         sem_out.at[par])

    def owait(ck, par):
        i0 = i_base + ck * CHUNK
        pltpu.make_async_copy(obuf_v.at[par],
                              out_hbm.at[g, :, pl.ds(i0, CHUNK), :],
                              sem_out.at[par]).wait()

    istart(0, 0)
    istart(1, 1)

    @pl.loop(0, NCHUNK, step=2)
    def _chunk(ck0):
        for par in range(2):
            ck = ck0 + par
            iwait(ck, par)

            @pl.when(ck >= 2)
            def _():
                owait(ck - 2, par)

            GRP = 4

            for c in range(CHUNK):
                @plsc.parallel_loop(0, N // LANES, unroll=1)
                def _t(t):
                    sl = pl.ds(t * LANES, LANES)
                    spvec = sp_v[par, c, sl]
                    sptvec = spt_v[par, c, sl]
                    abvec = ab_v[par, c, sl]

                    def gathers(k0):
                        ks = range(k0, k0 + GRP)
                        fwds = [plsc.load_gather(encp_v.at[pl.ds(k * S, S)],
                                                 [spvec]) for k in ks]
                        revs = [plsc.load_gather(encrp_v.at[pl.ds(k * S, S)],
                                                 [sptvec]) for k in ks]
                        return fwds, revs

                    def arith(k0, fwds, revs):
                        for u, k in enumerate(range(k0, k0 + GRP)):
                            pair = (plsc.bitcast(fwds[u], jnp.bfloat16)
                                    + plsc.bitcast(revs[u], jnp.bfloat16))
                            lo, hi = plsc.unpack(
                                pair, format=plsc.PackFormat.INTERLEAVED)
                            obuf_v[par, 2 * k, c, sl] = lo + abvec
                            obuf_v[par, 2 * k + 1, c, sl] = hi + abvec

                    pend = gathers(0)
                    for k0 in range(GRP, HP, GRP):
                        cur = gathers(k0)
                        arith(k0 - GRP, *pend)
                        pend = cur
                    arith(HP - GRP, *pend)

            ostart(ck, par)

            @pl.when(ck + 2 < NCHUNK)
            def _():
                istart(ck + 2, par)

    owait(NCHUNK - 2, 0)
    owait(NCHUNK - 1, 1)


@jax.jit
def kernel(attn_bias, spatial_pos, spatial_pos_encoder, spatial_pos_encoder_rev):
    spt = _transpose_gnn(spatial_pos)

    mesh = plsc.VectorSubcoreMesh(core_axis_name="c", subcore_axis_name="s")
    run = pl.kernel(
        _sc_body,
        out_type=jax.ShapeDtypeStruct((G, H, N, N), jnp.float32),
        mesh=mesh,
        compiler_params=pltpu.CompilerParams(needs_layout_passes=False),
        scratch_types=[
            pltpu.VMEM((HP * S,), jnp.int32),
            pltpu.VMEM((HP * S,), jnp.int32),
            pltpu.VMEM((2, CHUNK, N), jnp.int32),
            pltpu.VMEM((2, CHUNK, N), jnp.int32),
            pltpu.VMEM((2, CHUNK, N), jnp.float32),
            pltpu.VMEM((2, H, CHUNK, N), jnp.float32),

            pltpu.SemaphoreType.DMA((2,)),
            pltpu.SemaphoreType.DMA((2,)),
        ],
    )
    return run(attn_bias, spatial_pos, spt,
               _pack_table(spatial_pos_encoder),
               _pack_table(spatial_pos_encoder_rev))
